# Initial kernel scaffold; baseline (speedup 1.0000x reference)
import jax
import jax.numpy as jnp
from jax.experimental import pallas as pl


def kernel(head, rel, tail, entity_table, relation_table):
    raise NotImplementedError("write your pallas kernel here")


# speedup vs baseline: 1.5523x; 1.5523x over previous
import dataclasses
import functools

import jax
import jax.numpy as jnp
from jax import lax
from jax.experimental import pallas as pl
from jax.experimental.pallas import tpu as pltpu
from jax.experimental.pallas import tpu_sc as plsc

NC = 2
NS = 16
L = 16
NW = NC * NS


def kernel(head, rel, tail, entity_table, relation_table):
    B = head.shape[0]
    TWO_D = entity_table.shape[1]
    D = TWO_D // 2
    assert B % (8 * NW) == 0 and D % L == 0
    b_per_w = B // NW

    mesh = plsc.VectorSubcoreMesh(core_axis_name="c", subcore_axis_name="s")
    cp = pltpu.CompilerParams()
    if "needs_layout_passes" in pltpu.CompilerParams.__dataclass_fields__:
        cp = dataclasses.replace(cp, needs_layout_passes=False)

    @functools.partial(
        pl.kernel,
        mesh=mesh,
        compiler_params=cp,
        out_type=jax.ShapeDtypeStruct((B,), jnp.float32),
        scratch_types=[
            pltpu.VMEM((b_per_w,), jnp.int32),
            pltpu.VMEM((b_per_w,), jnp.int32),
            pltpu.VMEM((b_per_w,), jnp.int32),
            pltpu.VMEM((b_per_w, TWO_D), jnp.float32),
            pltpu.VMEM((b_per_w, TWO_D), jnp.float32),
            pltpu.VMEM((b_per_w, TWO_D), jnp.float32),
            pltpu.VMEM((b_per_w,), jnp.float32),
            pltpu.SemaphoreType.DMA,
        ],
    )
    def score_kernel(head_hbm, rel_hbm, tail_hbm, etab_hbm, rtab_hbm, out_hbm,
                     hidx_v, ridx_v, tidx_v, h_v, r_v, t_v, s_v, sem):
        wid = lax.axis_index("s") * NC + lax.axis_index("c")
        base = wid * b_per_w
        pltpu.sync_copy(head_hbm.at[pl.ds(base, b_per_w)], hidx_v)
        pltpu.sync_copy(rel_hbm.at[pl.ds(base, b_per_w)], ridx_v)
        pltpu.sync_copy(tail_hbm.at[pl.ds(base, b_per_w)], tidx_v)
        gh = pltpu.async_copy(etab_hbm.at[hidx_v], h_v, sem)
        gr = pltpu.async_copy(rtab_hbm.at[ridx_v], r_v, sem)
        gt = pltpu.async_copy(etab_hbm.at[tidx_v], t_v, sem)
        gh.wait()
        gr.wait()
        gt.wait()

        lane_id = lax.iota(jnp.int32, L)

        @pl.loop(0, b_per_w, step=L)
        def _(g):
            scores = jnp.zeros((L,), jnp.float32)
            for j in range(L):
                row = g + j
                acc = jnp.zeros((L,), jnp.float32)
                for c in range(D // L):
                    dr = pl.ds(c * L, L)
                    di = pl.ds(D + c * L, L)
                    hr = h_v[row, dr]
                    hi = h_v[row, di]
                    rr = r_v[row, dr]
                    ri = r_v[row, di]
                    tr = t_v[row, dr]
                    ti = t_v[row, di]
                    acc = acc + hr * (rr * tr + ri * ti) + hi * (rr * ti - ri * tr)
                s = jnp.sum(acc)
                scores = jnp.where(lane_id == j, s, scores)
            s_v[pl.ds(g, L)] = scores

        pltpu.sync_copy(s_v, out_hbm.at[pl.ds(base, b_per_w)])

    return score_kernel(head, rel, tail, entity_table, relation_table)
